# Initial kernel scaffold; baseline (speedup 1.0000x reference)
import functools

import jax
import jax.numpy as jnp
from jax import lax
from jax.experimental import pallas as pl
from jax.experimental.pallas import tpu as pltpu
from jax.experimental.pallas import tpu_sc as plsc

N = 10000
E = 320000
IN_DIM = 128
HID = 16
H0 = 8
H1 = 1
C = 40

F32 = jnp.float32
HI = jax.lax.Precision.HIGHEST

ROW_BLK = 400
GRID = N // ROW_BLK
CHUNK = 128
NW = 32
NEG = -1e30


def _tc1_body(x_ref, w_ref, alp_ref, arp_ref, t_ref, er_ref, ml_ref, mr_ref):
    feat = jnp.dot(x_ref[...], w_ref[...], preferred_element_type=F32,
                   precision=HI)
    pad = jnp.where(lax.broadcasted_iota(jnp.int32, (1, 16), 1) < H0,
                    0.0, NEG).astype(F32)
    elp = jnp.dot(feat, alp_ref[...], preferred_element_type=F32,
                  precision=HI) + pad
    erp = jnp.dot(feat, arp_ref[...], preferred_element_type=F32,
                  precision=HI)
    t_ref[:, :IN_DIM] = feat
    t_ref[:, IN_DIM:] = elp

    er_ref[...] = erp

    @pl.when(pl.program_id(0) == 0)
    def _():
        ml_ref[...] = jnp.full((1, 16), NEG, F32)
        mr_ref[...] = jnp.full((1, 16), NEG, F32)

    ml_ref[...] = jnp.maximum(ml_ref[...], jnp.max(elp, axis=0, keepdims=True))
    mr_ref[...] = jnp.maximum(mr_ref[...], jnp.max(erp, axis=0, keepdims=True))


def _tc1(x, w0, alp, arp):
    return pl.pallas_call(
        _tc1_body,
        grid=(GRID,),
        in_specs=[
            pl.BlockSpec((ROW_BLK, IN_DIM), lambda i: (i, 0)),
            pl.BlockSpec((IN_DIM, IN_DIM), lambda i: (0, 0)),
            pl.BlockSpec((IN_DIM, 16), lambda i: (0, 0)),
            pl.BlockSpec((IN_DIM, 16), lambda i: (0, 0)),
        ],
        out_specs=[
            pl.BlockSpec((ROW_BLK, IN_DIM + 16), lambda i: (i, 0)),
            pl.BlockSpec((ROW_BLK, 16), lambda i: (i, 0)),
            pl.BlockSpec((1, 16), lambda i: (0, 0)),
            pl.BlockSpec((1, 16), lambda i: (0, 0)),
        ],
        out_shape=[
            jax.ShapeDtypeStruct((N, IN_DIM + 16), F32),
            jax.ShapeDtypeStruct((N, 16), F32),
            jax.ShapeDtypeStruct((1, 16), F32),
            jax.ShapeDtypeStruct((1, 16), F32),
        ],
    )(x, w0, alp, arp)


def _bcast_lane(v, h):
    idx = jnp.full((16, 1), h, jnp.int32)
    dn = lax.GatherDimensionNumbers(offset_dims=(), collapsed_slice_dims=(0,),
                                    start_index_map=(0,))
    return lax.gather(v, idx, dn, (1,),
                      mode=lax.GatherScatterMode.PROMISE_IN_BOUNDS)


def _make_edge_pass(row_w, feat_w, heads):
    n_chunks = E // CHUNK
    n_iter = (n_chunks + NW - 1) // NW
    rows_per_tile = N // 16
    mesh = plsc.VectorSubcoreMesh(core_axis_name="c", subcore_axis_name="s")

    @functools.partial(
        pl.kernel,
        mesh=mesh,
        out_type=jax.ShapeDtypeStruct((2, N, row_w), F32),
        scratch_types=[
            pltpu.VMEM((CHUNK,), jnp.int32),
            pltpu.VMEM((CHUNK,), jnp.int32),
            pltpu.VMEM((CHUNK, row_w), F32),
            pltpu.VMEM((CHUNK, 16), F32),
            pltpu.VMEM((16,), F32),
            pltpu.VMEM((16,), F32),
            pltpu.VMEM_SHARED((N, row_w), F32),
            pltpu.SemaphoreType.DMA,
            pltpu.SemaphoreType.DMA,
        ],
    )
    def edge_pass(src_hbm, dst_hbm, t_hbm, er_hbm, ml_hbm, mr_hbm, z_hbm,
                  out_hbm, sidx, didx, gbuf, erbuf, mlv, mrv, acc_sh,
                  sem1, sem2):
        core = lax.axis_index("c")
        sid = lax.axis_index("s")
        wid = core * 16 + sid
        r0 = sid * rows_per_tile

        pltpu.sync_copy(z_hbm.at[pl.ds(r0, rows_per_tile)],
                        acc_sh.at[pl.ds(r0, rows_per_tile)])
        pltpu.sync_copy(ml_hbm, mlv)
        pltpu.sync_copy(mr_hbm, mrv)
        plsc.subcore_barrier()
        m = mlv[...] + mrv[...]

        @pl.loop(0, n_iter)
        def _(j):
            cid = wid + j * NW

            @pl.when(cid < n_chunks)
            def _():
                base = cid * CHUNK
                pltpu.sync_copy(src_hbm.at[pl.ds(base, CHUNK)], sidx)
                pltpu.sync_copy(dst_hbm.at[pl.ds(base, CHUNK)], didx)
                cp1 = pltpu.async_copy(t_hbm.at[sidx], gbuf, sem1)
                cp2 = pltpu.async_copy(er_hbm.at[didx], erbuf, sem2)
                cp1.wait()
                cp2.wait()

                @pl.loop(0, CHUNK)
                def _(k):
                    a = gbuf[k, pl.ds(feat_w, 16)] + erbuf[k, :]
                    a = jnp.maximum(a, 0.2 * a)
                    w = jnp.exp(jnp.minimum(a - m, 0.0))
                    gbuf[k, pl.ds(feat_w, 16)] = w
                    for g, h in enumerate(heads):
                        wh = _bcast_lane(w, h)
                        gbuf[k, pl.ds(16 * g, 16)] = (
                            gbuf[k, pl.ds(16 * g, 16)] * wh)

                pltpu.sync_copy(gbuf, acc_sh.at[didx], add=True)

        plsc.subcore_barrier()
        pltpu.sync_copy(acc_sh.at[pl.ds(r0, rows_per_tile)],
                        out_hbm.at[core, pl.ds(r0, rows_per_tile)])

    return edge_pass


_edge_pass0 = _make_edge_pass(IN_DIM + 16, IN_DIM, tuple(range(H0)))
_edge_pass1 = _make_edge_pass(64, 48, (0, 0, 0))


def _tc2_body(p0_ref, p1_ref, b_ref, h_ref, s_ref, sq_ref):
    acc = p0_ref[...] + p1_ref[...]
    featacc = acc[:, :IN_DIM]
    wsum = acc[:, IN_DIM:IN_DIM + H0]
    expand = jnp.where(
        lax.broadcasted_iota(jnp.int32, (H0, IN_DIM), 1) // HID
        == lax.broadcasted_iota(jnp.int32, (H0, IN_DIM), 0),
        1.0, 0.0).astype(F32)
    wexp = jnp.dot(wsum, expand, preferred_element_type=F32, precision=HI)
    h = featacc / (wexp + 1e-30) + b_ref[...]
    h_ref[...] = h

    @pl.when(pl.program_id(0) == 0)
    def _():
        s_ref[...] = jnp.zeros((1, IN_DIM), F32)
        sq_ref[...] = jnp.zeros((1, IN_DIM), F32)

    s_ref[...] += jnp.sum(h, axis=0, keepdims=True)
    sq_ref[...] += jnp.sum(h * h, axis=0, keepdims=True)


def _tc2(p0, p1, b0):
    return pl.pallas_call(
        _tc2_body,
        grid=(GRID,),
        in_specs=[
            pl.BlockSpec((ROW_BLK, IN_DIM + 16), lambda i: (i, 0)),
            pl.BlockSpec((ROW_BLK, IN_DIM + 16), lambda i: (i, 0)),
            pl.BlockSpec((1, IN_DIM), lambda i: (0, 0)),
        ],
        out_specs=[
            pl.BlockSpec((ROW_BLK, IN_DIM), lambda i: (i, 0)),
            pl.BlockSpec((1, IN_DIM), lambda i: (0, 0)),
            pl.BlockSpec((1, IN_DIM), lambda i: (0, 0)),
        ],
        out_shape=[
            jax.ShapeDtypeStruct((N, IN_DIM), F32),
            jax.ShapeDtypeStruct((1, IN_DIM), F32),
            jax.ShapeDtypeStruct((1, IN_DIM), F32),
        ],
    )(p0, p1, b0)


def _tc3_body(h_ref, s_ref, sq_ref, g_ref, be_ref, w1_ref, alp_ref, arp_ref,
              t_ref, er_ref, ml_ref, mr_ref):
    mean = s_ref[...] / N
    var = sq_ref[...] / N - mean * mean
    hn = (h_ref[...] - mean) * lax.rsqrt(var + 1e-5) * g_ref[...] + be_ref[...]
    hn = jnp.maximum(hn, 0.0)
    feat = jnp.dot(hn, w1_ref[...], preferred_element_type=F32, precision=HI)
    pad = jnp.where(lax.broadcasted_iota(jnp.int32, (1, 16), 1) < H1,
                    0.0, NEG).astype(F32)
    elp = jnp.dot(feat, alp_ref[...], preferred_element_type=F32,
                  precision=HI) + pad
    erp = jnp.dot(feat, arp_ref[...], preferred_element_type=F32,
                  precision=HI)
    t_ref[:, :48] = feat
    t_ref[:, 48:] = elp
    er_ref[...] = erp

    @pl.when(pl.program_id(0) == 0)
    def _():
        ml_ref[...] = jnp.full((1, 16), NEG, F32)
        mr_ref[...] = jnp.full((1, 16), NEG, F32)

    ml_ref[...] = jnp.maximum(ml_ref[...], jnp.max(elp, axis=0, keepdims=True))
    mr_ref[...] = jnp.maximum(mr_ref[...], jnp.max(erp, axis=0, keepdims=True))


def _tc3(h, s, sq, gamma, beta, w1p, alp, arp):
    return pl.pallas_call(
        _tc3_body,
        grid=(GRID,),
        in_specs=[
            pl.BlockSpec((ROW_BLK, IN_DIM), lambda i: (i, 0)),
            pl.BlockSpec((1, IN_DIM), lambda i: (0, 0)),
            pl.BlockSpec((1, IN_DIM), lambda i: (0, 0)),
            pl.BlockSpec((1, IN_DIM), lambda i: (0, 0)),
            pl.BlockSpec((1, IN_DIM), lambda i: (0, 0)),
            pl.BlockSpec((IN_DIM, 48), lambda i: (0, 0)),
            pl.BlockSpec((48, 16), lambda i: (0, 0)),
            pl.BlockSpec((48, 16), lambda i: (0, 0)),
        ],
        out_specs=[
            pl.BlockSpec((ROW_BLK, 64), lambda i: (i, 0)),
            pl.BlockSpec((ROW_BLK, 16), lambda i: (i, 0)),
            pl.BlockSpec((1, 16), lambda i: (0, 0)),
            pl.BlockSpec((1, 16), lambda i: (0, 0)),
        ],
        out_shape=[
            jax.ShapeDtypeStruct((N, 64), F32),
            jax.ShapeDtypeStruct((N, 16), F32),
            jax.ShapeDtypeStruct((1, 16), F32),
            jax.ShapeDtypeStruct((1, 16), F32),
        ],
    )(h, s, sq, gamma, beta, w1p, alp, arp)


def _tc4_body(p0_ref, p1_ref, b_ref, o_ref):
    acc = p0_ref[...] + p1_ref[...]
    feat = acc[:, :C]
    wsum = acc[:, 48:49]
    logits = feat / (wsum + 1e-30) + b_ref[...]
    mx = jnp.max(logits, axis=1, keepdims=True)
    ex = jnp.exp(logits - mx)
    lse = jnp.log(jnp.sum(ex, axis=1, keepdims=True))
    o_ref[...] = logits - mx - lse


def _tc4(p0, p1, b1):
    return pl.pallas_call(
        _tc4_body,
        grid=(GRID,),
        in_specs=[
            pl.BlockSpec((ROW_BLK, 64), lambda i: (i, 0)),
            pl.BlockSpec((ROW_BLK, 64), lambda i: (i, 0)),
            pl.BlockSpec((1, C), lambda i: (0, 0)),
        ],
        out_specs=pl.BlockSpec((ROW_BLK, C), lambda i: (i, 0)),
        out_shape=jax.ShapeDtypeStruct((N, C), F32),
    )(p0, p1, b1)


def kernel(x, edge_index, W0, attn_l0, attn_r0, bias0, gamma, beta,
           W1, attn_l1, attn_r1, bias1):
    src = edge_index[0]
    dst = edge_index[1]

    onehot0 = (jnp.arange(IN_DIM)[:, None] // HID
               == jnp.arange(16)[None, :]).astype(F32)
    alp0 = attn_l0.reshape(-1)[:, None] * onehot0
    arp0 = attn_r0.reshape(-1)[:, None] * onehot0

    w1p = jnp.pad(W1, ((0, 0), (0, 8)))
    col1 = (jnp.arange(16)[None, :] == 0).astype(F32)
    alp1 = jnp.pad(attn_l1.reshape(-1), (0, 8))[:, None] * col1
    arp1 = jnp.pad(attn_r1.reshape(-1), (0, 8))[:, None] * col1

    t0, er0, ml0, mr0 = _tc1(x, W0, alp0, arp0)
    z0 = jnp.zeros((N, IN_DIM + 16), F32)
    parts0 = _edge_pass0(src, dst, t0, er0, ml0.reshape(16), mr0.reshape(16),
                         z0)
    h0, s0, sq0 = _tc2(parts0[0], parts0[1], bias0.reshape(1, IN_DIM))
    t1, er1, ml1, mr1 = _tc3(h0, s0, sq0, gamma.reshape(1, IN_DIM),
                             beta.reshape(1, IN_DIM), w1p, alp1, arp1)
    z1 = jnp.zeros((N, 64), F32)
    parts1 = _edge_pass1(src, dst, t1, er1, ml1.reshape(16), mr1.reshape(16),
                         z1)
    return _tc4(parts1[0], parts1[1], bias1.reshape(1, C))


# speedup vs baseline: 48.5805x; 48.5805x over previous
import functools

import jax
import jax.numpy as jnp
from jax import lax
from jax.experimental import pallas as pl
from jax.experimental.pallas import tpu as pltpu
from jax.experimental.pallas import tpu_sc as plsc

N = 10000
E = 320000
IN_DIM = 128
HID = 16
H0 = 8
H1 = 1
C = 40

F32 = jnp.float32
HI = jax.lax.Precision.HIGHEST

ROW_BLK = 400
GRID = N // ROW_BLK
CHUNK = 128
NW = 32
NEG = -1e30


def _tc1_body(x_ref, w_ref, alp_ref, arp_ref, t_ref, er_ref, ml_ref, mr_ref):
    feat = jnp.dot(x_ref[...], w_ref[...], preferred_element_type=F32,
                   precision=HI)
    pad = jnp.where(lax.broadcasted_iota(jnp.int32, (1, 16), 1) < H0,
                    0.0, NEG).astype(F32)
    elp = jnp.dot(feat, alp_ref[...], preferred_element_type=F32,
                  precision=HI) + pad
    erp = jnp.dot(feat, arp_ref[...], preferred_element_type=F32,
                  precision=HI)
    t_ref[:, :IN_DIM] = feat
    t_ref[:, IN_DIM:] = elp

    er_ref[...] = erp

    @pl.when(pl.program_id(0) == 0)
    def _():
        ml_ref[...] = jnp.full((1, 16), NEG, F32)
        mr_ref[...] = jnp.full((1, 16), NEG, F32)

    ml_ref[...] = jnp.maximum(ml_ref[...], jnp.max(elp, axis=0, keepdims=True))
    mr_ref[...] = jnp.maximum(mr_ref[...], jnp.max(erp, axis=0, keepdims=True))


def _tc1(x, w0, alp, arp):
    return pl.pallas_call(
        _tc1_body,
        grid=(GRID,),
        in_specs=[
            pl.BlockSpec((ROW_BLK, IN_DIM), lambda i: (i, 0)),
            pl.BlockSpec((IN_DIM, IN_DIM), lambda i: (0, 0)),
            pl.BlockSpec((IN_DIM, 16), lambda i: (0, 0)),
            pl.BlockSpec((IN_DIM, 16), lambda i: (0, 0)),
        ],
        out_specs=[
            pl.BlockSpec((ROW_BLK, IN_DIM + 16), lambda i: (i, 0)),
            pl.BlockSpec((ROW_BLK, 16), lambda i: (i, 0)),
            pl.BlockSpec((1, 16), lambda i: (0, 0)),
            pl.BlockSpec((1, 16), lambda i: (0, 0)),
        ],
        out_shape=[
            jax.ShapeDtypeStruct((N, IN_DIM + 16), F32),
            jax.ShapeDtypeStruct((N, 16), F32),
            jax.ShapeDtypeStruct((1, 16), F32),
            jax.ShapeDtypeStruct((1, 16), F32),
        ],
    )(x, w0, alp, arp)


def _bcast_lane(v, h):
    idx = jnp.full((16, 1), h, jnp.int32)
    dn = lax.GatherDimensionNumbers(offset_dims=(), collapsed_slice_dims=(0,),
                                    start_index_map=(0,))
    return lax.gather(v, idx, dn, (1,),
                      mode=lax.GatherScatterMode.PROMISE_IN_BOUNDS)


def _make_edge_pass(row_w, feat_w, heads):
    n_chunks = E // CHUNK
    n_iter = (n_chunks + NW - 1) // NW
    rows_per_tile = 624
    tail_rows = N - 16 * rows_per_tile
    mesh = plsc.VectorSubcoreMesh(core_axis_name="c", subcore_axis_name="s")

    @functools.partial(
        pl.kernel,
        mesh=mesh,
        compiler_params=pltpu.CompilerParams(use_tc_tiling_on_sc=False),
        out_type=jax.ShapeDtypeStruct((2, N, row_w), F32),
        scratch_types=[
            pltpu.VMEM((CHUNK,), jnp.int32),
            pltpu.VMEM((CHUNK,), jnp.int32),
            pltpu.VMEM((CHUNK, row_w), F32),
            pltpu.VMEM((CHUNK, 16), F32),
            pltpu.VMEM((16,), F32),
            pltpu.VMEM((16,), F32),
            pltpu.VMEM_SHARED((N, row_w), F32),
            pltpu.SemaphoreType.DMA,
            pltpu.SemaphoreType.DMA,
        ],
    )
    def edge_pass(src_hbm, dst_hbm, t_hbm, er_hbm, ml_hbm, mr_hbm, z_hbm,
                  out_hbm, sidx, didx, gbuf, erbuf, mlv, mrv, acc_sh,
                  sem1, sem2):
        core = lax.axis_index("c")
        sid = lax.axis_index("s")
        wid = core * 16 + sid
        r0 = sid * rows_per_tile

        pltpu.sync_copy(z_hbm.at[pl.ds(r0, rows_per_tile)],
                        acc_sh.at[pl.ds(r0, rows_per_tile)])

        @pl.when(sid == 15)
        def _():
            pltpu.sync_copy(z_hbm.at[pl.ds(16 * rows_per_tile, tail_rows)],
                            acc_sh.at[pl.ds(16 * rows_per_tile, tail_rows)])

        pltpu.sync_copy(ml_hbm, mlv)
        pltpu.sync_copy(mr_hbm, mrv)
        plsc.subcore_barrier()
        m = mlv[...] + mrv[...]

        @pl.loop(0, n_iter)
        def _(j):
            cid = wid + j * NW

            @pl.when(cid < n_chunks)
            def _():
                base = cid * CHUNK
                pltpu.sync_copy(src_hbm.at[pl.ds(base, CHUNK)], sidx)
                pltpu.sync_copy(dst_hbm.at[pl.ds(base, CHUNK)], didx)
                cp1 = pltpu.async_copy(t_hbm.at[sidx], gbuf, sem1)
                cp2 = pltpu.async_copy(er_hbm.at[didx], erbuf, sem2)
                cp1.wait()
                cp2.wait()

                @pl.loop(0, CHUNK)
                def _(k):
                    a = gbuf[k, pl.ds(feat_w, 16)] + erbuf[k, :]
                    a = jnp.maximum(a, 0.2 * a)
                    w = jnp.exp(jnp.minimum(a - m, 0.0))
                    gbuf[k, pl.ds(feat_w, 16)] = w
                    for g, h in enumerate(heads):
                        wh = _bcast_lane(w, h)
                        gbuf[k, pl.ds(16 * g, 16)] = (
                            gbuf[k, pl.ds(16 * g, 16)] * wh)

                pltpu.sync_copy(gbuf, acc_sh.at[didx], add=True)

        plsc.subcore_barrier()
        pltpu.sync_copy(acc_sh.at[pl.ds(r0, rows_per_tile)],
                        out_hbm.at[core, pl.ds(r0, rows_per_tile)])

        @pl.when(sid == 15)
        def _():
            pltpu.sync_copy(
                acc_sh.at[pl.ds(16 * rows_per_tile, tail_rows)],
                out_hbm.at[core, pl.ds(16 * rows_per_tile, tail_rows)])

    return edge_pass


_edge_pass0 = _make_edge_pass(IN_DIM + 16, IN_DIM, tuple(range(H0)))
_edge_pass1 = _make_edge_pass(64, 48, (0, 0, 0))


def _tc2_body(p0_ref, p1_ref, b_ref, h_ref, s_ref, sq_ref):
    acc = p0_ref[...] + p1_ref[...]
    featacc = acc[:, :IN_DIM]
    wsum = acc[:, IN_DIM:IN_DIM + H0]
    expand = jnp.where(
        lax.broadcasted_iota(jnp.int32, (H0, IN_DIM), 1) // HID
        == lax.broadcasted_iota(jnp.int32, (H0, IN_DIM), 0),
        1.0, 0.0).astype(F32)
    wexp = jnp.dot(wsum, expand, preferred_element_type=F32, precision=HI)
    h = featacc / (wexp + 1e-30) + b_ref[...]
    h_ref[...] = h

    @pl.when(pl.program_id(0) == 0)
    def _():
        s_ref[...] = jnp.zeros((1, IN_DIM), F32)
        sq_ref[...] = jnp.zeros((1, IN_DIM), F32)

    s_ref[...] += jnp.sum(h, axis=0, keepdims=True)
    sq_ref[...] += jnp.sum(h * h, axis=0, keepdims=True)


def _tc2(p0, p1, b0):
    return pl.pallas_call(
        _tc2_body,
        grid=(GRID,),
        in_specs=[
            pl.BlockSpec((ROW_BLK, IN_DIM + 16), lambda i: (i, 0)),
            pl.BlockSpec((ROW_BLK, IN_DIM + 16), lambda i: (i, 0)),
            pl.BlockSpec((1, IN_DIM), lambda i: (0, 0)),
        ],
        out_specs=[
            pl.BlockSpec((ROW_BLK, IN_DIM), lambda i: (i, 0)),
            pl.BlockSpec((1, IN_DIM), lambda i: (0, 0)),
            pl.BlockSpec((1, IN_DIM), lambda i: (0, 0)),
        ],
        out_shape=[
            jax.ShapeDtypeStruct((N, IN_DIM), F32),
            jax.ShapeDtypeStruct((1, IN_DIM), F32),
            jax.ShapeDtypeStruct((1, IN_DIM), F32),
        ],
    )(p0, p1, b0)


def _tc3_body(h_ref, s_ref, sq_ref, g_ref, be_ref, w1_ref, alp_ref, arp_ref,
              t_ref, er_ref, ml_ref, mr_ref):
    mean = s_ref[...] / N
    var = sq_ref[...] / N - mean * mean
    hn = (h_ref[...] - mean) * lax.rsqrt(var + 1e-5) * g_ref[...] + be_ref[...]
    hn = jnp.maximum(hn, 0.0)
    feat = jnp.dot(hn, w1_ref[...], preferred_element_type=F32, precision=HI)
    pad = jnp.where(lax.broadcasted_iota(jnp.int32, (1, 16), 1) < H1,
                    0.0, NEG).astype(F32)
    elp = jnp.dot(feat, alp_ref[...], preferred_element_type=F32,
                  precision=HI) + pad
    erp = jnp.dot(feat, arp_ref[...], preferred_element_type=F32,
                  precision=HI)
    t_ref[:, :48] = feat
    t_ref[:, 48:] = elp
    er_ref[...] = erp

    @pl.when(pl.program_id(0) == 0)
    def _():
        ml_ref[...] = jnp.full((1, 16), NEG, F32)
        mr_ref[...] = jnp.full((1, 16), NEG, F32)

    ml_ref[...] = jnp.maximum(ml_ref[...], jnp.max(elp, axis=0, keepdims=True))
    mr_ref[...] = jnp.maximum(mr_ref[...], jnp.max(erp, axis=0, keepdims=True))


def _tc3(h, s, sq, gamma, beta, w1p, alp, arp):
    return pl.pallas_call(
        _tc3_body,
        grid=(GRID,),
        in_specs=[
            pl.BlockSpec((ROW_BLK, IN_DIM), lambda i: (i, 0)),
            pl.BlockSpec((1, IN_DIM), lambda i: (0, 0)),
            pl.BlockSpec((1, IN_DIM), lambda i: (0, 0)),
            pl.BlockSpec((1, IN_DIM), lambda i: (0, 0)),
            pl.BlockSpec((1, IN_DIM), lambda i: (0, 0)),
            pl.BlockSpec((IN_DIM, 48), lambda i: (0, 0)),
            pl.BlockSpec((48, 16), lambda i: (0, 0)),
            pl.BlockSpec((48, 16), lambda i: (0, 0)),
        ],
        out_specs=[
            pl.BlockSpec((ROW_BLK, 64), lambda i: (i, 0)),
            pl.BlockSpec((ROW_BLK, 16), lambda i: (i, 0)),
            pl.BlockSpec((1, 16), lambda i: (0, 0)),
            pl.BlockSpec((1, 16), lambda i: (0, 0)),
        ],
        out_shape=[
            jax.ShapeDtypeStruct((N, 64), F32),
            jax.ShapeDtypeStruct((N, 16), F32),
            jax.ShapeDtypeStruct((1, 16), F32),
            jax.ShapeDtypeStruct((1, 16), F32),
        ],
    )(h, s, sq, gamma, beta, w1p, alp, arp)


def _tc4_body(p0_ref, p1_ref, b_ref, o_ref):
    acc = p0_ref[...] + p1_ref[...]
    feat = acc[:, :C]
    wsum = acc[:, 48:49]
    logits = feat / (wsum + 1e-30) + b_ref[...]
    mx = jnp.max(logits, axis=1, keepdims=True)
    ex = jnp.exp(logits - mx)
    lse = jnp.log(jnp.sum(ex, axis=1, keepdims=True))
    o_ref[...] = logits - mx - lse


def _tc4(p0, p1, b1):
    return pl.pallas_call(
        _tc4_body,
        grid=(GRID,),
        in_specs=[
            pl.BlockSpec((ROW_BLK, 64), lambda i: (i, 0)),
            pl.BlockSpec((ROW_BLK, 64), lambda i: (i, 0)),
            pl.BlockSpec((1, C), lambda i: (0, 0)),
        ],
        out_specs=pl.BlockSpec((ROW_BLK, C), lambda i: (i, 0)),
        out_shape=jax.ShapeDtypeStruct((N, C), F32),
    )(p0, p1, b1)


def kernel(x, edge_index, W0, attn_l0, attn_r0, bias0, gamma, beta,
           W1, attn_l1, attn_r1, bias1):
    src = edge_index[0]
    dst = edge_index[1]

    onehot0 = (jnp.arange(IN_DIM)[:, None] // HID
               == jnp.arange(16)[None, :]).astype(F32)
    alp0 = attn_l0.reshape(-1)[:, None] * onehot0
    arp0 = attn_r0.reshape(-1)[:, None] * onehot0

    w1p = jnp.pad(W1, ((0, 0), (0, 8)))
    col1 = (jnp.arange(16)[None, :] == 0).astype(F32)
    alp1 = jnp.pad(attn_l1.reshape(-1), (0, 8))[:, None] * col1
    arp1 = jnp.pad(attn_r1.reshape(-1), (0, 8))[:, None] * col1

    t0, er0, ml0, mr0 = _tc1(x, W0, alp0, arp0)
    z0 = jnp.zeros((N, IN_DIM + 16), F32)
    parts0 = _edge_pass0(src, dst, t0, er0, ml0.reshape(16), mr0.reshape(16),
                         z0)
    h0, s0, sq0 = _tc2(parts0[0], parts0[1], bias0.reshape(1, IN_DIM))
    t1, er1, ml1, mr1 = _tc3(h0, s0, sq0, gamma.reshape(1, IN_DIM),
                             beta.reshape(1, IN_DIM), w1p, alp1, arp1)
    z1 = jnp.zeros((N, 64), F32)
    parts1 = _edge_pass1(src, dst, t1, er1, ml1.reshape(16), mr1.reshape(16),
                         z1)
    return _tc4(parts1[0], parts1[1], bias1.reshape(1, C))


# speedup vs baseline: 82.1129x; 1.6902x over previous
import functools

import jax
import jax.numpy as jnp
from jax import lax
from jax.experimental import pallas as pl
from jax.experimental.pallas import tpu as pltpu
from jax.experimental.pallas import tpu_sc as plsc

N = 10000
E = 320000
IN_DIM = 128
HID = 16
H0 = 8
H1 = 1
C = 40

F32 = jnp.float32
HI = jax.lax.Precision.HIGHEST

ROW_BLK = 400
GRID = N // ROW_BLK
CHUNK = 128
NW = 32
NEG = -1e30


def _tc1_body(x_ref, w_ref, alp_ref, arp_ref, t_ref, er_ref, ml_ref, mr_ref):
    feat = jnp.dot(x_ref[...], w_ref[...], preferred_element_type=F32,
                   precision=HI)
    pad = jnp.where(lax.broadcasted_iota(jnp.int32, (1, 16), 1) < H0,
                    0.0, NEG).astype(F32)
    elp = jnp.dot(feat, alp_ref[...], preferred_element_type=F32,
                  precision=HI) + pad
    erp = jnp.dot(feat, arp_ref[...], preferred_element_type=F32,
                  precision=HI)
    t_ref[:, :IN_DIM] = feat
    t_ref[:, IN_DIM:] = elp

    er_ref[...] = erp

    @pl.when(pl.program_id(0) == 0)
    def _():
        ml_ref[...] = jnp.full((1, 16), NEG, F32)
        mr_ref[...] = jnp.full((1, 16), NEG, F32)

    ml_ref[...] = jnp.maximum(ml_ref[...], jnp.max(elp, axis=0, keepdims=True))
    mr_ref[...] = jnp.maximum(mr_ref[...], jnp.max(erp, axis=0, keepdims=True))


def _tc1(x, w0, alp, arp):
    return pl.pallas_call(
        _tc1_body,
        grid=(GRID,),
        in_specs=[
            pl.BlockSpec((ROW_BLK, IN_DIM), lambda i: (i, 0)),
            pl.BlockSpec((IN_DIM, IN_DIM), lambda i: (0, 0)),
            pl.BlockSpec((IN_DIM, 16), lambda i: (0, 0)),
            pl.BlockSpec((IN_DIM, 16), lambda i: (0, 0)),
        ],
        out_specs=[
            pl.BlockSpec((ROW_BLK, IN_DIM + 16), lambda i: (i, 0)),
            pl.BlockSpec((ROW_BLK, 16), lambda i: (i, 0)),
            pl.BlockSpec((1, 16), lambda i: (0, 0)),
            pl.BlockSpec((1, 16), lambda i: (0, 0)),
        ],
        out_shape=[
            jax.ShapeDtypeStruct((N, IN_DIM + 16), F32),
            jax.ShapeDtypeStruct((N, 16), F32),
            jax.ShapeDtypeStruct((1, 16), F32),
            jax.ShapeDtypeStruct((1, 16), F32),
        ],
    )(x, w0, alp, arp)


def _bcast_lane(v, h):
    idx = jnp.full((16, 1), h, jnp.int32)
    dn = lax.GatherDimensionNumbers(offset_dims=(), collapsed_slice_dims=(0,),
                                    start_index_map=(0,))
    return lax.gather(v, idx, dn, (1,),
                      mode=lax.GatherScatterMode.PROMISE_IN_BOUNDS)


def _make_edge_pass(row_w, feat_w, heads, unroll):
    n_chunks = E // CHUNK
    base_c = n_chunks // NW
    extra = n_chunks % NW
    max_c = base_c + (1 if extra else 0)
    n_pair = (max_c + 1) // 2
    rows_per_tile = 624
    tail_rows = N - 16 * rows_per_tile
    mesh = plsc.VectorSubcoreMesh(core_axis_name="c", subcore_axis_name="s")

    @functools.partial(
        pl.kernel,
        mesh=mesh,
        compiler_params=pltpu.CompilerParams(use_tc_tiling_on_sc=False),
        out_type=jax.ShapeDtypeStruct((2, N, row_w), F32),
        scratch_types=[
            pltpu.VMEM((2, CHUNK), jnp.int32),
            pltpu.VMEM((2, CHUNK), jnp.int32),
            pltpu.VMEM((CHUNK, row_w), F32),
            pltpu.VMEM((CHUNK, row_w), F32),
            pltpu.VMEM((CHUNK, 16), F32),
            pltpu.VMEM((16,), F32),
            pltpu.VMEM((16,), F32),
            pltpu.VMEM_SHARED((N, row_w), F32),
            pltpu.SemaphoreType.DMA,
            pltpu.SemaphoreType.DMA,
        ],
    )
    def edge_pass(sd_hbm, t_hbm, er_hbm, ml_hbm, mr_hbm, z_hbm,
                  out_hbm, sd0, sd1, gbuf0, gbuf1, erbuf,
                  mlv, mrv, acc_sh, sg0, sg1):
        core = lax.axis_index("c")
        sid = lax.axis_index("s")
        wid = core * 16 + sid
        r0 = sid * rows_per_tile
        c_lo = wid * base_c + jnp.minimum(wid, extra)
        n_my = base_c + jnp.where(wid < extra, 1, 0)

        pltpu.sync_copy(z_hbm.at[pl.ds(r0, rows_per_tile)],
                        acc_sh.at[pl.ds(r0, rows_per_tile)])

        @pl.when(sid == 15)
        def _():
            pltpu.sync_copy(z_hbm.at[pl.ds(16 * rows_per_tile, tail_rows)],
                            acc_sh.at[pl.ds(16 * rows_per_tile, tail_rows)])

        pltpu.sync_copy(ml_hbm, mlv)
        pltpu.sync_copy(mr_hbm, mrv)
        plsc.subcore_barrier()
        m = mlv[...] + mrv[...]

        gbufs = (gbuf0, gbuf1)
        sds = (sd0, sd1)
        gsems = (sg0, sg1)

        def load_sd_and_start_gather(j, b):
            pltpu.sync_copy(sd_hbm.at[j], sds[b])
            pltpu.async_copy(t_hbm.at[sds[b].at[0]], gbufs[b], gsems[b])

        def process(j, b):
            gbuf = gbufs[b]
            sd = sds[b]
            pltpu.sync_copy(er_hbm.at[sd.at[1]], erbuf)
            pltpu.make_async_copy(t_hbm.at[sd.at[0]], gbuf, gsems[b]).wait()

            @plsc.parallel_loop(0, CHUNK, 1, unroll=unroll)
            def _(k):
                a = gbuf[k, pl.ds(feat_w, 16)] + erbuf[k, :]
                a = jnp.maximum(a, 0.2 * a)
                w = jnp.exp(jnp.minimum(a - m, 0.0))
                gbuf[k, pl.ds(feat_w, 16)] = w
                whs = {}
                for g, h in enumerate(heads):
                    if h not in whs:
                        whs[h] = _bcast_lane(w, h)
                    gbuf[k, pl.ds(16 * g, 16)] = (
                        gbuf[k, pl.ds(16 * g, 16)] * whs[h])

            pltpu.sync_copy(gbuf, acc_sh.at[sd.at[1]], add=True)

            @pl.when(j + 2 < n_my)
            def _():
                load_sd_and_start_gather(c_lo + j + 2, b)

        load_sd_and_start_gather(c_lo, 0)

        @pl.when(1 < n_my)
        def _():
            load_sd_and_start_gather(c_lo + 1, 1)

        @pl.loop(0, n_pair)
        def _(p):
            j0 = p * 2

            @pl.when(j0 < n_my)
            def _():
                process(j0, 0)

            @pl.when(j0 + 1 < n_my)
            def _():
                process(j0 + 1, 1)

        plsc.subcore_barrier()
        pltpu.sync_copy(acc_sh.at[pl.ds(r0, rows_per_tile)],
                        out_hbm.at[core, pl.ds(r0, rows_per_tile)])

        @pl.when(sid == 15)
        def _():
            pltpu.sync_copy(
                acc_sh.at[pl.ds(16 * rows_per_tile, tail_rows)],
                out_hbm.at[core, pl.ds(16 * rows_per_tile, tail_rows)])

    return edge_pass


_edge_pass0 = _make_edge_pass(IN_DIM + 16, IN_DIM, tuple(range(H0)), 2)
_edge_pass1 = _make_edge_pass(64, 48, (0, 0, 0), 2)


def _tc2_body(p0_ref, p1_ref, b_ref, h_ref, s_ref, sq_ref):
    acc = p0_ref[...] + p1_ref[...]
    featacc = acc[:, :IN_DIM]
    wsum = acc[:, IN_DIM:IN_DIM + H0]
    expand = jnp.where(
        lax.broadcasted_iota(jnp.int32, (H0, IN_DIM), 1) // HID
        == lax.broadcasted_iota(jnp.int32, (H0, IN_DIM), 0),
        1.0, 0.0).astype(F32)
    wexp = jnp.dot(wsum, expand, preferred_element_type=F32, precision=HI)
    h = featacc / (wexp + 1e-30) + b_ref[...]
    h_ref[...] = h

    @pl.when(pl.program_id(0) == 0)
    def _():
        s_ref[...] = jnp.zeros((1, IN_DIM), F32)
        sq_ref[...] = jnp.zeros((1, IN_DIM), F32)

    s_ref[...] += jnp.sum(h, axis=0, keepdims=True)
    sq_ref[...] += jnp.sum(h * h, axis=0, keepdims=True)


def _tc2(p0, p1, b0):
    return pl.pallas_call(
        _tc2_body,
        grid=(GRID,),
        in_specs=[
            pl.BlockSpec((ROW_BLK, IN_DIM + 16), lambda i: (i, 0)),
            pl.BlockSpec((ROW_BLK, IN_DIM + 16), lambda i: (i, 0)),
            pl.BlockSpec((1, IN_DIM), lambda i: (0, 0)),
        ],
        out_specs=[
            pl.BlockSpec((ROW_BLK, IN_DIM), lambda i: (i, 0)),
            pl.BlockSpec((1, IN_DIM), lambda i: (0, 0)),
            pl.BlockSpec((1, IN_DIM), lambda i: (0, 0)),
        ],
        out_shape=[
            jax.ShapeDtypeStruct((N, IN_DIM), F32),
            jax.ShapeDtypeStruct((1, IN_DIM), F32),
            jax.ShapeDtypeStruct((1, IN_DIM), F32),
        ],
    )(p0, p1, b0)


def _tc3_body(h_ref, s_ref, sq_ref, g_ref, be_ref, w1_ref, alp_ref, arp_ref,
              t_ref, er_ref, ml_ref, mr_ref):
    mean = s_ref[...] / N
    var = sq_ref[...] / N - mean * mean
    hn = (h_ref[...] - mean) * lax.rsqrt(var + 1e-5) * g_ref[...] + be_ref[...]
    hn = jnp.maximum(hn, 0.0)
    feat = jnp.dot(hn, w1_ref[...], preferred_element_type=F32, precision=HI)
    pad = jnp.where(lax.broadcasted_iota(jnp.int32, (1, 16), 1) < H1,
                    0.0, NEG).astype(F32)
    elp = jnp.dot(feat, alp_ref[...], preferred_element_type=F32,
                  precision=HI) + pad
    erp = jnp.dot(feat, arp_ref[...], preferred_element_type=F32,
                  precision=HI)
    t_ref[:, :48] = feat
    t_ref[:, 48:] = elp
    er_ref[...] = erp

    @pl.when(pl.program_id(0) == 0)
    def _():
        ml_ref[...] = jnp.full((1, 16), NEG, F32)
        mr_ref[...] = jnp.full((1, 16), NEG, F32)

    ml_ref[...] = jnp.maximum(ml_ref[...], jnp.max(elp, axis=0, keepdims=True))
    mr_ref[...] = jnp.maximum(mr_ref[...], jnp.max(erp, axis=0, keepdims=True))


def _tc3(h, s, sq, gamma, beta, w1p, alp, arp):
    return pl.pallas_call(
        _tc3_body,
        grid=(GRID,),
        in_specs=[
            pl.BlockSpec((ROW_BLK, IN_DIM), lambda i: (i, 0)),
            pl.BlockSpec((1, IN_DIM), lambda i: (0, 0)),
            pl.BlockSpec((1, IN_DIM), lambda i: (0, 0)),
            pl.BlockSpec((1, IN_DIM), lambda i: (0, 0)),
            pl.BlockSpec((1, IN_DIM), lambda i: (0, 0)),
            pl.BlockSpec((IN_DIM, 48), lambda i: (0, 0)),
            pl.BlockSpec((48, 16), lambda i: (0, 0)),
            pl.BlockSpec((48, 16), lambda i: (0, 0)),
        ],
        out_specs=[
            pl.BlockSpec((ROW_BLK, 64), lambda i: (i, 0)),
            pl.BlockSpec((ROW_BLK, 16), lambda i: (i, 0)),
            pl.BlockSpec((1, 16), lambda i: (0, 0)),
            pl.BlockSpec((1, 16), lambda i: (0, 0)),
        ],
        out_shape=[
            jax.ShapeDtypeStruct((N, 64), F32),
            jax.ShapeDtypeStruct((N, 16), F32),
            jax.ShapeDtypeStruct((1, 16), F32),
            jax.ShapeDtypeStruct((1, 16), F32),
        ],
    )(h, s, sq, gamma, beta, w1p, alp, arp)


def _tc4_body(p0_ref, p1_ref, b_ref, o_ref):
    acc = p0_ref[...] + p1_ref[...]
    feat = acc[:, :C]
    wsum = acc[:, 48:49]
    logits = feat / (wsum + 1e-30) + b_ref[...]
    mx = jnp.max(logits, axis=1, keepdims=True)
    ex = jnp.exp(logits - mx)
    lse = jnp.log(jnp.sum(ex, axis=1, keepdims=True))
    o_ref[...] = logits - mx - lse


def _tc4(p0, p1, b1):
    return pl.pallas_call(
        _tc4_body,
        grid=(GRID,),
        in_specs=[
            pl.BlockSpec((ROW_BLK, 64), lambda i: (i, 0)),
            pl.BlockSpec((ROW_BLK, 64), lambda i: (i, 0)),
            pl.BlockSpec((1, C), lambda i: (0, 0)),
        ],
        out_specs=pl.BlockSpec((ROW_BLK, C), lambda i: (i, 0)),
        out_shape=jax.ShapeDtypeStruct((N, C), F32),
    )(p0, p1, b1)


def kernel(x, edge_index, W0, attn_l0, attn_r0, bias0, gamma, beta,
           W1, attn_l1, attn_r1, bias1):
    sd = jnp.stack([edge_index[0].reshape(-1, CHUNK),
                    edge_index[1].reshape(-1, CHUNK)], axis=1)

    onehot0 = (jnp.arange(IN_DIM)[:, None] // HID
               == jnp.arange(16)[None, :]).astype(F32)
    alp0 = attn_l0.reshape(-1)[:, None] * onehot0
    arp0 = attn_r0.reshape(-1)[:, None] * onehot0

    w1p = jnp.pad(W1, ((0, 0), (0, 8)))
    col1 = (jnp.arange(16)[None, :] == 0).astype(F32)
    alp1 = jnp.pad(attn_l1.reshape(-1), (0, 8))[:, None] * col1
    arp1 = jnp.pad(attn_r1.reshape(-1), (0, 8))[:, None] * col1

    t0, er0, ml0, mr0 = _tc1(x, W0, alp0, arp0)
    z0 = jnp.zeros((N, IN_DIM + 16), F32)
    parts0 = _edge_pass0(sd, t0, er0, ml0.reshape(16), mr0.reshape(16), z0)
    h0, s0, sq0 = _tc2(parts0[0], parts0[1], bias0.reshape(1, IN_DIM))
    t1, er1, ml1, mr1 = _tc3(h0, s0, sq0, gamma.reshape(1, IN_DIM),
                             beta.reshape(1, IN_DIM), w1p, alp1, arp1)
    z1 = jnp.zeros((N, 64), F32)
    parts1 = _edge_pass1(sd, t1, er1, ml1.reshape(16), mr1.reshape(16), z1)
    return _tc4(parts1[0], parts1[1], bias1.reshape(1, C))


# speedup vs baseline: 107.5313x; 1.3096x over previous
import functools

import jax
import jax.numpy as jnp
from jax import lax
from jax.experimental import pallas as pl
from jax.experimental.pallas import tpu as pltpu
from jax.experimental.pallas import tpu_sc as plsc

N = 10000
E = 320000
IN_DIM = 128
HID = 16
H0 = 8
H1 = 1
C = 40

F32 = jnp.float32
HI = jax.lax.Precision.HIGHEST

ROW_BLK = 400
GRID = N // ROW_BLK
CHUNK = 64
NW = 32
NEG = -1e30


def _tc1_body(x_ref, w_ref, alp_ref, arp_ref, t_ref, er_ref, ml_ref, mr_ref):
    feat = jnp.dot(x_ref[...], w_ref[...], preferred_element_type=F32,
                   precision=HI)
    pad = jnp.where(lax.broadcasted_iota(jnp.int32, (1, 16), 1) < H0,
                    0.0, NEG).astype(F32)
    elp = jnp.dot(feat, alp_ref[...], preferred_element_type=F32,
                  precision=HI) + pad
    erp = jnp.dot(feat, arp_ref[...], preferred_element_type=F32,
                  precision=HI)
    t_ref[:, :IN_DIM] = feat
    t_ref[:, IN_DIM:] = elp

    er_ref[...] = erp

    @pl.when(pl.program_id(0) == 0)
    def _():
        ml_ref[...] = jnp.full((1, 16), NEG, F32)
        mr_ref[...] = jnp.full((1, 16), NEG, F32)

    ml_ref[...] = jnp.maximum(ml_ref[...], jnp.max(elp, axis=0, keepdims=True))
    mr_ref[...] = jnp.maximum(mr_ref[...], jnp.max(erp, axis=0, keepdims=True))


def _tc1(x, w0, alp, arp):
    return pl.pallas_call(
        _tc1_body,
        grid=(GRID,),
        in_specs=[
            pl.BlockSpec((ROW_BLK, IN_DIM), lambda i: (i, 0)),
            pl.BlockSpec((IN_DIM, IN_DIM), lambda i: (0, 0)),
            pl.BlockSpec((IN_DIM, 16), lambda i: (0, 0)),
            pl.BlockSpec((IN_DIM, 16), lambda i: (0, 0)),
        ],
        out_specs=[
            pl.BlockSpec((ROW_BLK, IN_DIM + 16), lambda i: (i, 0)),
            pl.BlockSpec((ROW_BLK, 16), lambda i: (i, 0)),
            pl.BlockSpec((1, 16), lambda i: (0, 0)),
            pl.BlockSpec((1, 16), lambda i: (0, 0)),
        ],
        out_shape=[
            jax.ShapeDtypeStruct((N, IN_DIM + 16), F32),
            jax.ShapeDtypeStruct((N, 16), F32),
            jax.ShapeDtypeStruct((1, 16), F32),
            jax.ShapeDtypeStruct((1, 16), F32),
        ],
    )(x, w0, alp, arp)


def _bcast_lane(v, h):
    idx = jnp.full((16, 1), h, jnp.int32)
    dn = lax.GatherDimensionNumbers(offset_dims=(), collapsed_slice_dims=(0,),
                                    start_index_map=(0,))
    return lax.gather(v, idx, dn, (1,),
                      mode=lax.GatherScatterMode.PROMISE_IN_BOUNDS)


def _make_edge_pass(row_w, feat_w, heads, unroll):
    n_chunks = E // CHUNK
    base_c = n_chunks // NW
    extra = n_chunks % NW
    max_c = base_c + (1 if extra else 0)
    UNR = 12
    n_iter = (max_c + UNR - 1) // UNR
    rows_per_tile = 624
    tail_rows = N - 16 * rows_per_tile
    mesh = plsc.VectorSubcoreMesh(core_axis_name="c", subcore_axis_name="s")

    @functools.partial(
        pl.kernel,
        mesh=mesh,
        compiler_params=pltpu.CompilerParams(use_tc_tiling_on_sc=False),
        out_type=jax.ShapeDtypeStruct((2, N, row_w), F32),
        scratch_types=[
            pltpu.VMEM((4, 2, CHUNK), jnp.int32),
            pltpu.VMEM((3, CHUNK, row_w), F32),
            pltpu.VMEM((3, CHUNK, 16), F32),
            pltpu.VMEM((16,), F32),
            pltpu.VMEM((16,), F32),
            pltpu.VMEM_SHARED((N, row_w), F32),
            pltpu.SemaphoreType.DMA,
            pltpu.SemaphoreType.DMA,
            pltpu.SemaphoreType.DMA,
            pltpu.SemaphoreType.DMA,
            pltpu.SemaphoreType.DMA,
            pltpu.SemaphoreType.DMA,
            pltpu.SemaphoreType.DMA,
            pltpu.SemaphoreType.DMA,
            pltpu.SemaphoreType.DMA,
            pltpu.SemaphoreType.DMA,
        ],
    )
    def edge_pass(sd_hbm, t_hbm, er_hbm, ml_hbm, mr_hbm, z_hbm,
                  out_hbm, sd, gb, eb, mlv, mrv, acc_sh,
                  sd_s0, sd_s1, sd_s2, sd_s3, g_s0, g_s1, g_s2,
                  s_s0, s_s1, s_s2):
        core = lax.axis_index("c")
        sid = lax.axis_index("s")
        wid = core * 16 + sid
        r0 = sid * rows_per_tile
        c_lo = wid * base_c + jnp.minimum(wid, extra)
        n_my = base_c + jnp.where(wid < extra, 1, 0)

        pltpu.sync_copy(z_hbm.at[pl.ds(r0, rows_per_tile)],
                        acc_sh.at[pl.ds(r0, rows_per_tile)])

        @pl.when(sid == 15)
        def _():
            pltpu.sync_copy(z_hbm.at[pl.ds(16 * rows_per_tile, tail_rows)],
                            acc_sh.at[pl.ds(16 * rows_per_tile, tail_rows)])

        pltpu.sync_copy(ml_hbm, mlv)
        pltpu.sync_copy(mr_hbm, mrv)
        plsc.subcore_barrier()
        m = mlv[...] + mrv[...]

        sdsems = (sd_s0, sd_s1, sd_s2, sd_s3)
        gsems = (g_s0, g_s1, g_s2)
        ssems = (s_s0, s_s1, s_s2)

        def issue_sd(j, b4):
            @pl.when(j < n_my)
            def _():
                pltpu.async_copy(sd_hbm.at[c_lo + j], sd.at[b4], sdsems[b4])

        def wait_sd(b4):
            pltpu.make_async_copy(sd_hbm.at[0], sd.at[b4], sdsems[b4]).wait()

        def issue_gather(j, b3, b4):
            @pl.when(j < n_my)
            def _():
                wait_sd(b4)
                pltpu.async_copy(t_hbm.at[sd.at[b4, 0]], gb.at[b3], gsems[b3])
                pltpu.async_copy(er_hbm.at[sd.at[b4, 1]], eb.at[b3], gsems[b3])

        def wait_gather(b3):
            pltpu.make_async_copy(t_hbm.at[sd.at[0, 0]], gb.at[b3],
                                  gsems[b3]).wait()
            pltpu.make_async_copy(er_hbm.at[sd.at[0, 1]], eb.at[b3],
                                  gsems[b3]).wait()

        def wait_scatter(b3):
            pltpu.make_async_copy(gb.at[b3], acc_sh.at[sd.at[0, 1]],
                                  ssems[b3]).wait()

        def process(j, jj):
            b3 = jj % 3
            b4 = jj % 4
            gbuf = gb.at[b3]
            erbuf = eb.at[b3]

            @pl.when(j < n_my)
            def _():
                wait_gather(b3)

                @plsc.parallel_loop(0, CHUNK, 1, unroll=unroll)
                def _(k):
                    a = gbuf[k, pl.ds(feat_w, 16)] + erbuf[k, :]
                    a = jnp.maximum(a, 0.2 * a)
                    w = jnp.exp(jnp.minimum(a - m, 0.0))
                    gbuf[k, pl.ds(feat_w, 16)] = w
                    whs = {}
                    for g, h in enumerate(heads):
                        if h not in whs:
                            whs[h] = _bcast_lane(w, h)
                        gbuf[k, pl.ds(16 * g, 16)] = (
                            gbuf[k, pl.ds(16 * g, 16)] * whs[h])

                @pl.when(j >= 1)
                def _():
                    wait_scatter((jj + 2) % 3)

                issue_sd(j + 3, (jj + 3) % 4)
                pltpu.async_copy(gbuf, acc_sh.at[sd.at[b4, 1]], ssems[b3],
                                 add=True)
                issue_gather(j + 2, (jj + 2) % 3, (jj + 2) % 4)

        issue_sd(0, 0)
        issue_sd(1, 1)
        issue_sd(2, 2)
        issue_gather(0, 0, 0)
        issue_gather(1, 1, 1)

        @pl.loop(0, n_iter)
        def _(p):
            j0 = p * UNR
            for r in range(UNR):
                process(j0 + r, r)

        last = lax.rem(n_my - 1, 3)
        for b in range(3):
            @pl.when(last == b)
            def _():
                wait_scatter(b)

        plsc.subcore_barrier()
        pltpu.sync_copy(acc_sh.at[pl.ds(r0, rows_per_tile)],
                        out_hbm.at[core, pl.ds(r0, rows_per_tile)])

        @pl.when(sid == 15)
        def _():
            pltpu.sync_copy(
                acc_sh.at[pl.ds(16 * rows_per_tile, tail_rows)],
                out_hbm.at[core, pl.ds(16 * rows_per_tile, tail_rows)])

    return edge_pass


_edge_pass0 = _make_edge_pass(IN_DIM + 16, IN_DIM, tuple(range(H0)), 2)
_edge_pass1 = _make_edge_pass(64, 48, (0, 0, 0), 2)


def _tc2_body(p0_ref, p1_ref, b_ref, h_ref, s_ref, sq_ref):
    acc = p0_ref[...] + p1_ref[...]
    featacc = acc[:, :IN_DIM]
    wsum = acc[:, IN_DIM:IN_DIM + H0]
    expand = jnp.where(
        lax.broadcasted_iota(jnp.int32, (H0, IN_DIM), 1) // HID
        == lax.broadcasted_iota(jnp.int32, (H0, IN_DIM), 0),
        1.0, 0.0).astype(F32)
    wexp = jnp.dot(wsum, expand, preferred_element_type=F32, precision=HI)
    h = featacc / (wexp + 1e-30) + b_ref[...]
    h_ref[...] = h

    @pl.when(pl.program_id(0) == 0)
    def _():
        s_ref[...] = jnp.zeros((1, IN_DIM), F32)
        sq_ref[...] = jnp.zeros((1, IN_DIM), F32)

    s_ref[...] += jnp.sum(h, axis=0, keepdims=True)
    sq_ref[...] += jnp.sum(h * h, axis=0, keepdims=True)


def _tc2(p0, p1, b0):
    return pl.pallas_call(
        _tc2_body,
        grid=(GRID,),
        in_specs=[
            pl.BlockSpec((ROW_BLK, IN_DIM + 16), lambda i: (i, 0)),
            pl.BlockSpec((ROW_BLK, IN_DIM + 16), lambda i: (i, 0)),
            pl.BlockSpec((1, IN_DIM), lambda i: (0, 0)),
        ],
        out_specs=[
            pl.BlockSpec((ROW_BLK, IN_DIM), lambda i: (i, 0)),
            pl.BlockSpec((1, IN_DIM), lambda i: (0, 0)),
            pl.BlockSpec((1, IN_DIM), lambda i: (0, 0)),
        ],
        out_shape=[
            jax.ShapeDtypeStruct((N, IN_DIM), F32),
            jax.ShapeDtypeStruct((1, IN_DIM), F32),
            jax.ShapeDtypeStruct((1, IN_DIM), F32),
        ],
    )(p0, p1, b0)


def _tc3_body(h_ref, s_ref, sq_ref, g_ref, be_ref, w1_ref, alp_ref, arp_ref,
              t_ref, er_ref, ml_ref, mr_ref):
    mean = s_ref[...] / N
    var = sq_ref[...] / N - mean * mean
    hn = (h_ref[...] - mean) * lax.rsqrt(var + 1e-5) * g_ref[...] + be_ref[...]
    hn = jnp.maximum(hn, 0.0)
    feat = jnp.dot(hn, w1_ref[...], preferred_element_type=F32, precision=HI)
    pad = jnp.where(lax.broadcasted_iota(jnp.int32, (1, 16), 1) < H1,
                    0.0, NEG).astype(F32)
    elp = jnp.dot(feat, alp_ref[...], preferred_element_type=F32,
                  precision=HI) + pad
    erp = jnp.dot(feat, arp_ref[...], preferred_element_type=F32,
                  precision=HI)
    t_ref[:, :48] = feat
    t_ref[:, 48:] = elp
    er_ref[...] = erp

    @pl.when(pl.program_id(0) == 0)
    def _():
        ml_ref[...] = jnp.full((1, 16), NEG, F32)
        mr_ref[...] = jnp.full((1, 16), NEG, F32)

    ml_ref[...] = jnp.maximum(ml_ref[...], jnp.max(elp, axis=0, keepdims=True))
    mr_ref[...] = jnp.maximum(mr_ref[...], jnp.max(erp, axis=0, keepdims=True))


def _tc3(h, s, sq, gamma, beta, w1p, alp, arp):
    return pl.pallas_call(
        _tc3_body,
        grid=(GRID,),
        in_specs=[
            pl.BlockSpec((ROW_BLK, IN_DIM), lambda i: (i, 0)),
            pl.BlockSpec((1, IN_DIM), lambda i: (0, 0)),
            pl.BlockSpec((1, IN_DIM), lambda i: (0, 0)),
            pl.BlockSpec((1, IN_DIM), lambda i: (0, 0)),
            pl.BlockSpec((1, IN_DIM), lambda i: (0, 0)),
            pl.BlockSpec((IN_DIM, 48), lambda i: (0, 0)),
            pl.BlockSpec((48, 16), lambda i: (0, 0)),
            pl.BlockSpec((48, 16), lambda i: (0, 0)),
        ],
        out_specs=[
            pl.BlockSpec((ROW_BLK, 64), lambda i: (i, 0)),
            pl.BlockSpec((ROW_BLK, 16), lambda i: (i, 0)),
            pl.BlockSpec((1, 16), lambda i: (0, 0)),
            pl.BlockSpec((1, 16), lambda i: (0, 0)),
        ],
        out_shape=[
            jax.ShapeDtypeStruct((N, 64), F32),
            jax.ShapeDtypeStruct((N, 16), F32),
            jax.ShapeDtypeStruct((1, 16), F32),
            jax.ShapeDtypeStruct((1, 16), F32),
        ],
    )(h, s, sq, gamma, beta, w1p, alp, arp)


def _tc4_body(p0_ref, p1_ref, b_ref, o_ref):
    acc = p0_ref[...] + p1_ref[...]
    feat = acc[:, :C]
    wsum = acc[:, 48:49]
    logits = feat / (wsum + 1e-30) + b_ref[...]
    mx = jnp.max(logits, axis=1, keepdims=True)
    ex = jnp.exp(logits - mx)
    lse = jnp.log(jnp.sum(ex, axis=1, keepdims=True))
    o_ref[...] = logits - mx - lse


def _tc4(p0, p1, b1):
    return pl.pallas_call(
        _tc4_body,
        grid=(GRID,),
        in_specs=[
            pl.BlockSpec((ROW_BLK, 64), lambda i: (i, 0)),
            pl.BlockSpec((ROW_BLK, 64), lambda i: (i, 0)),
            pl.BlockSpec((1, C), lambda i: (0, 0)),
        ],
        out_specs=pl.BlockSpec((ROW_BLK, C), lambda i: (i, 0)),
        out_shape=jax.ShapeDtypeStruct((N, C), F32),
    )(p0, p1, b1)


def kernel(x, edge_index, W0, attn_l0, attn_r0, bias0, gamma, beta,
           W1, attn_l1, attn_r1, bias1):
    sd = jnp.stack([edge_index[0].reshape(-1, CHUNK),
                    edge_index[1].reshape(-1, CHUNK)], axis=1)

    onehot0 = (jnp.arange(IN_DIM)[:, None] // HID
               == jnp.arange(16)[None, :]).astype(F32)
    alp0 = attn_l0.reshape(-1)[:, None] * onehot0
    arp0 = attn_r0.reshape(-1)[:, None] * onehot0

    w1p = jnp.pad(W1, ((0, 0), (0, 8)))
    col1 = (jnp.arange(16)[None, :] == 0).astype(F32)
    alp1 = jnp.pad(attn_l1.reshape(-1), (0, 8))[:, None] * col1
    arp1 = jnp.pad(attn_r1.reshape(-1), (0, 8))[:, None] * col1

    t0, er0, ml0, mr0 = _tc1(x, W0, alp0, arp0)
    z0 = jnp.zeros((N, IN_DIM + 16), F32)
    parts0 = _edge_pass0(sd, t0, er0, ml0.reshape(16), mr0.reshape(16), z0)
    h0, s0, sq0 = _tc2(parts0[0], parts0[1], bias0.reshape(1, IN_DIM))
    t1, er1, ml1, mr1 = _tc3(h0, s0, sq0, gamma.reshape(1, IN_DIM),
                             beta.reshape(1, IN_DIM), w1p, alp1, arp1)
    z1 = jnp.zeros((N, 64), F32)
    parts1 = _edge_pass1(sd, t1, er1, ml1.reshape(16), mr1.reshape(16), z1)
    return _tc4(parts1[0], parts1[1], bias1.reshape(1, C))


# speedup vs baseline: 145.1090x; 1.3495x over previous
import functools

import jax
import jax.numpy as jnp
from jax import lax
from jax.experimental import pallas as pl
from jax.experimental.pallas import tpu as pltpu
from jax.experimental.pallas import tpu_sc as plsc

N = 10000
E = 320000
IN_DIM = 128
HID = 16
H0 = 8
H1 = 1
C = 40

F32 = jnp.float32
HI = jax.lax.Precision.DEFAULT

ROW_BLK = 1000
GRID = N // ROW_BLK
NW = 32
NEG = -1e30


def _tc1_body(x_ref, w_ref, alp_ref, arp_ref, t_ref, er_ref, ml_ref, mr_ref):
    feat = jnp.dot(x_ref[...], w_ref[...], preferred_element_type=F32,
                   precision=HI)
    pad = jnp.where(lax.broadcasted_iota(jnp.int32, (1, 16), 1) < H0,
                    0.0, NEG).astype(F32)
    elp = jnp.dot(feat, alp_ref[...], preferred_element_type=F32,
                  precision=HI) + pad
    erp = jnp.dot(feat, arp_ref[...], preferred_element_type=F32,
                  precision=HI)
    t_ref[:, :IN_DIM] = feat
    t_ref[:, IN_DIM:] = elp

    er_ref[...] = erp

    @pl.when(pl.program_id(0) == 0)
    def _():
        ml_ref[...] = jnp.full((1, 16), NEG, F32)
        mr_ref[...] = jnp.full((1, 16), NEG, F32)

    ml_ref[...] = jnp.maximum(ml_ref[...], jnp.max(elp, axis=0, keepdims=True))
    mr_ref[...] = jnp.maximum(mr_ref[...], jnp.max(erp, axis=0, keepdims=True))


def _tc1(x, w0, alp, arp):
    return pl.pallas_call(
        _tc1_body,
        grid=(GRID,),
        in_specs=[
            pl.BlockSpec((ROW_BLK, IN_DIM), lambda i: (i, 0)),
            pl.BlockSpec((IN_DIM, IN_DIM), lambda i: (0, 0)),
            pl.BlockSpec((IN_DIM, 16), lambda i: (0, 0)),
            pl.BlockSpec((IN_DIM, 16), lambda i: (0, 0)),
        ],
        out_specs=[
            pl.BlockSpec((ROW_BLK, IN_DIM + 16), lambda i: (i, 0)),
            pl.BlockSpec((ROW_BLK, 16), lambda i: (i, 0)),
            pl.BlockSpec((1, 16), lambda i: (0, 0)),
            pl.BlockSpec((1, 16), lambda i: (0, 0)),
        ],
        out_shape=[
            jax.ShapeDtypeStruct((N, IN_DIM + 16), F32),
            jax.ShapeDtypeStruct((N, 16), F32),
            jax.ShapeDtypeStruct((1, 16), F32),
            jax.ShapeDtypeStruct((1, 16), F32),
        ],
    )(x, w0, alp, arp)


def _bcast_lane(v, h):
    idx = jnp.full((16, 1), h, jnp.int32)
    dn = lax.GatherDimensionNumbers(offset_dims=(), collapsed_slice_dims=(0,),
                                    start_index_map=(0,))
    return lax.gather(v, idx, dn, (1,),
                      mode=lax.GatherScatterMode.PROMISE_IN_BOUNDS)


def _make_edge_pass(row_w, feat_w, heads, chunk, unroll):
    n_chunks = E // chunk
    base_c = n_chunks // NW
    extra = n_chunks % NW
    max_c = base_c + (1 if extra else 0)
    UNR = 12
    n_iter = (max_c + UNR - 1) // UNR
    rows_per_tile = 624
    tail_rows = N - 16 * rows_per_tile
    mesh = plsc.VectorSubcoreMesh(core_axis_name="c", subcore_axis_name="s")
    out_sds = jax.ShapeDtypeStruct((N, row_w), F32)

    @functools.partial(
        pl.kernel,
        mesh=mesh,
        compiler_params=pltpu.CompilerParams(use_tc_tiling_on_sc=False),
        out_type=(out_sds, out_sds),
        scratch_types=[
            pltpu.VMEM((4, chunk), jnp.int32),
            pltpu.VMEM((4, chunk), jnp.int32),
            pltpu.VMEM((3, chunk, row_w), F32),
            pltpu.VMEM((3, chunk, 16), F32),
            pltpu.VMEM((1, 16), F32),
            pltpu.VMEM((1, 16), F32),
            pltpu.VMEM_SHARED((N, row_w), F32),
            pltpu.SemaphoreType.DMA,
            pltpu.SemaphoreType.DMA,
            pltpu.SemaphoreType.DMA,
            pltpu.SemaphoreType.DMA,
            pltpu.SemaphoreType.DMA,
            pltpu.SemaphoreType.DMA,
            pltpu.SemaphoreType.DMA,
            pltpu.SemaphoreType.DMA,
            pltpu.SemaphoreType.DMA,
            pltpu.SemaphoreType.DMA,
        ],
    )
    def edge_pass(ei_hbm, t_hbm, er_hbm, ml_hbm, mr_hbm, z_hbm,
                  o0_hbm, o1_hbm, sb, db, gb, eb, mlv, mrv, acc_sh,
                  i_s0, i_s1, i_s2, i_s3, g_s0, g_s1, g_s2,
                  s_s0, s_s1, s_s2):
        core = lax.axis_index("c")
        sid = lax.axis_index("s")
        wid = core * 16 + sid
        r0 = sid * rows_per_tile
        c_lo = wid * base_c + jnp.minimum(wid, extra)
        n_my = base_c + jnp.where(wid < extra, 1, 0)

        pltpu.sync_copy(z_hbm.at[pl.ds(r0, rows_per_tile)],
                        acc_sh.at[pl.ds(r0, rows_per_tile)])

        @pl.when(sid == 15)
        def _():
            pltpu.sync_copy(z_hbm.at[pl.ds(16 * rows_per_tile, tail_rows)],
                            acc_sh.at[pl.ds(16 * rows_per_tile, tail_rows)])

        pltpu.sync_copy(ml_hbm, mlv)
        pltpu.sync_copy(mr_hbm, mrv)
        plsc.subcore_barrier()
        m = mlv[0, :] + mrv[0, :]

        isems = (i_s0, i_s1, i_s2, i_s3)
        gsems = (g_s0, g_s1, g_s2)
        ssems = (s_s0, s_s1, s_s2)

        def issue_idx(j, b4):
            @pl.when(j < n_my)
            def _():
                e0 = (c_lo + j) * chunk
                pltpu.async_copy(ei_hbm.at[0, pl.ds(e0, chunk)], sb.at[b4],
                                 isems[b4])
                pltpu.async_copy(ei_hbm.at[1, pl.ds(e0, chunk)], db.at[b4],
                                 isems[b4])

        def wait_idx(b4):
            pltpu.make_async_copy(ei_hbm.at[0, pl.ds(0, chunk)], sb.at[b4],
                                  isems[b4]).wait()
            pltpu.make_async_copy(ei_hbm.at[1, pl.ds(0, chunk)], db.at[b4],
                                  isems[b4]).wait()

        def issue_gather(j, b3, b4):
            @pl.when(j < n_my)
            def _():
                wait_idx(b4)
                pltpu.async_copy(t_hbm.at[sb.at[b4]], gb.at[b3], gsems[b3])
                pltpu.async_copy(er_hbm.at[db.at[b4]], eb.at[b3], gsems[b3])

        def wait_gather(b3):
            pltpu.make_async_copy(t_hbm.at[sb.at[0]], gb.at[b3],
                                  gsems[b3]).wait()
            pltpu.make_async_copy(er_hbm.at[db.at[0]], eb.at[b3],
                                  gsems[b3]).wait()

        def wait_scatter(b3):
            pltpu.make_async_copy(gb.at[b3], acc_sh.at[db.at[0]],
                                  ssems[b3]).wait()

        def process(j, jj):
            b3 = jj % 3
            b4 = jj % 4
            gbuf = gb.at[b3]
            erbuf = eb.at[b3]

            @pl.when(j < n_my)
            def _():
                wait_gather(b3)

                @plsc.parallel_loop(0, chunk, 1, unroll=unroll)
                def _(k):
                    a = gbuf[k, pl.ds(feat_w, 16)] + erbuf[k, :]
                    a = jnp.maximum(a, 0.2 * a)
                    w = jnp.exp(jnp.minimum(a - m, 0.0))
                    gbuf[k, pl.ds(feat_w, 16)] = w
                    whs = {}
                    for g, h in enumerate(heads):
                        if h not in whs:
                            whs[h] = _bcast_lane(w, h)
                        gbuf[k, pl.ds(16 * g, 16)] = (
                            gbuf[k, pl.ds(16 * g, 16)] * whs[h])

                @pl.when(j >= 1)
                def _():
                    wait_scatter((jj + 2) % 3)

                issue_idx(j + 3, (jj + 3) % 4)
                pltpu.async_copy(gbuf, acc_sh.at[db.at[b4]], ssems[b3],
                                 add=True)
                issue_gather(j + 2, (jj + 2) % 3, (jj + 2) % 4)

        issue_idx(0, 0)
        issue_idx(1, 1)
        issue_idx(2, 2)
        issue_gather(0, 0, 0)
        issue_gather(1, 1, 1)

        @pl.loop(0, n_iter)
        def _(p):
            j0 = p * UNR
            for r in range(UNR):
                process(j0 + r, r)

        last = lax.rem(n_my - 1, 3)
        for b in range(3):
            @pl.when(last == b)
            def _():
                wait_scatter(b)

        plsc.subcore_barrier()

        @pl.when(core == 0)
        def _():
            pltpu.sync_copy(acc_sh.at[pl.ds(r0, rows_per_tile)],
                            o0_hbm.at[pl.ds(r0, rows_per_tile)])

            @pl.when(sid == 15)
            def _():
                pltpu.sync_copy(
                    acc_sh.at[pl.ds(16 * rows_per_tile, tail_rows)],
                    o0_hbm.at[pl.ds(16 * rows_per_tile, tail_rows)])

        @pl.when(core == 1)
        def _():
            pltpu.sync_copy(acc_sh.at[pl.ds(r0, rows_per_tile)],
                            o1_hbm.at[pl.ds(r0, rows_per_tile)])

            @pl.when(sid == 15)
            def _():
                pltpu.sync_copy(
                    acc_sh.at[pl.ds(16 * rows_per_tile, tail_rows)],
                    o1_hbm.at[pl.ds(16 * rows_per_tile, tail_rows)])

    return edge_pass


_edge_pass0 = _make_edge_pass(IN_DIM + 16, IN_DIM, tuple(range(H0)), 80, 2)
_edge_pass1 = _make_edge_pass(64, 48, (0, 0, 0), 128, 2)


def _tc23_body(p0_ref, p1_ref, b_ref, g_ref, be_ref, w1_ref, alp_ref, arp_ref,
               t_ref, er_ref, ml_ref, mr_ref, h_vmem, s_v, sq_v):
    ph = pl.program_id(0)
    i = pl.program_id(1)

    @pl.when(jnp.logical_and(ph == 0, i == 0))
    def _():
        s_v[...] = jnp.zeros((1, IN_DIM), F32)
        sq_v[...] = jnp.zeros((1, IN_DIM), F32)
        ml_ref[...] = jnp.full((1, 16), NEG, F32)
        mr_ref[...] = jnp.full((1, 16), NEG, F32)

    @pl.when(ph == 0)
    def _():
        acc = p0_ref[...] + p1_ref[...]
        featacc = acc[:, :IN_DIM]
        wsum = acc[:, IN_DIM:IN_DIM + H0]
        expand = jnp.where(
            lax.broadcasted_iota(jnp.int32, (H0, IN_DIM), 1) // HID
            == lax.broadcasted_iota(jnp.int32, (H0, IN_DIM), 0),
            1.0, 0.0).astype(F32)
        wexp = jnp.dot(wsum, expand, preferred_element_type=F32, precision=HI)
        h = featacc / (wexp + 1e-30) + b_ref[...]
        h_vmem[pl.ds(i * ROW_BLK, ROW_BLK), :] = h
        s_v[...] += jnp.sum(h, axis=0, keepdims=True)
        sq_v[...] += jnp.sum(h * h, axis=0, keepdims=True)

    @pl.when(ph == 1)
    def _():
        mean = s_v[...] / N
        var = sq_v[...] / N - mean * mean
        h = h_vmem[pl.ds(i * ROW_BLK, ROW_BLK), :]
        hn = (h - mean) * lax.rsqrt(var + 1e-5) * g_ref[...] + be_ref[...]
        hn = jnp.maximum(hn, 0.0)
        feat = jnp.dot(hn, w1_ref[...], preferred_element_type=F32,
                       precision=HI)
        pad = jnp.where(lax.broadcasted_iota(jnp.int32, (1, 16), 1) < H1,
                        0.0, NEG).astype(F32)
        elp = jnp.dot(feat, alp_ref[...], preferred_element_type=F32,
                      precision=HI) + pad
        erp = jnp.dot(feat, arp_ref[...], preferred_element_type=F32,
                      precision=HI)
        t_ref[:, :48] = feat
        t_ref[:, 48:] = elp
        er_ref[...] = erp
        ml_ref[...] = jnp.maximum(ml_ref[...],
                                  jnp.max(elp, axis=0, keepdims=True))
        mr_ref[...] = jnp.maximum(mr_ref[...],
                                  jnp.max(erp, axis=0, keepdims=True))


def _tc23(p0, p1, b0, gamma, beta, w1p, alp, arp):
    return pl.pallas_call(
        _tc23_body,
        grid=(2, GRID),
        in_specs=[
            pl.BlockSpec((ROW_BLK, IN_DIM + 16), lambda t, i: ((1 - t) * i, 0)),
            pl.BlockSpec((ROW_BLK, IN_DIM + 16), lambda t, i: ((1 - t) * i, 0)),
            pl.BlockSpec((1, IN_DIM), lambda t, i: (0, 0)),
            pl.BlockSpec((1, IN_DIM), lambda t, i: (0, 0)),
            pl.BlockSpec((1, IN_DIM), lambda t, i: (0, 0)),
            pl.BlockSpec((IN_DIM, 48), lambda t, i: (0, 0)),
            pl.BlockSpec((48, 16), lambda t, i: (0, 0)),
            pl.BlockSpec((48, 16), lambda t, i: (0, 0)),
        ],
        out_specs=[
            pl.BlockSpec((ROW_BLK, 64), lambda t, i: (t * i, 0)),
            pl.BlockSpec((ROW_BLK, 16), lambda t, i: (t * i, 0)),
            pl.BlockSpec((1, 16), lambda t, i: (0, 0)),
            pl.BlockSpec((1, 16), lambda t, i: (0, 0)),
        ],
        out_shape=[
            jax.ShapeDtypeStruct((N, 64), F32),
            jax.ShapeDtypeStruct((N, 16), F32),
            jax.ShapeDtypeStruct((1, 16), F32),
            jax.ShapeDtypeStruct((1, 16), F32),
        ],
        scratch_shapes=[
            pltpu.VMEM((N, IN_DIM), F32),
            pltpu.VMEM((1, IN_DIM), F32),
            pltpu.VMEM((1, IN_DIM), F32),
        ],
    )(p0, p1, b0, gamma, beta, w1p, alp, arp)


def _tc4_body(p0_ref, p1_ref, b_ref, o_ref):
    acc = p0_ref[...] + p1_ref[...]
    feat = acc[:, :C]
    wsum = acc[:, 48:49]
    logits = feat / (wsum + 1e-30) + b_ref[...]
    mx = jnp.max(logits, axis=1, keepdims=True)
    ex = jnp.exp(logits - mx)
    lse = jnp.log(jnp.sum(ex, axis=1, keepdims=True))
    o_ref[...] = logits - mx - lse


def _tc4(p0, p1, b1):
    return pl.pallas_call(
        _tc4_body,
        grid=(GRID,),
        in_specs=[
            pl.BlockSpec((ROW_BLK, 64), lambda i: (i, 0)),
            pl.BlockSpec((ROW_BLK, 64), lambda i: (i, 0)),
            pl.BlockSpec((1, C), lambda i: (0, 0)),
        ],
        out_specs=pl.BlockSpec((ROW_BLK, C), lambda i: (i, 0)),
        out_shape=jax.ShapeDtypeStruct((N, C), F32),
    )(p0, p1, b1)


def kernel(x, edge_index, W0, attn_l0, attn_r0, bias0, gamma, beta,
           W1, attn_l1, attn_r1, bias1):
    onehot0 = (jnp.arange(IN_DIM)[:, None] // HID
               == jnp.arange(16)[None, :]).astype(F32)
    alp0 = attn_l0.reshape(-1)[:, None] * onehot0
    arp0 = attn_r0.reshape(-1)[:, None] * onehot0

    w1p = jnp.pad(W1, ((0, 0), (0, 8)))
    col1 = (jnp.arange(16)[None, :] == 0).astype(F32)
    alp1 = jnp.pad(attn_l1.reshape(-1), (0, 8))[:, None] * col1
    arp1 = jnp.pad(attn_r1.reshape(-1), (0, 8))[:, None] * col1

    t0, er0, ml0, mr0 = _tc1(x, W0, alp0, arp0)
    z0 = jnp.zeros((N, IN_DIM + 16), F32)
    p00, p01 = _edge_pass0(edge_index, t0, er0, ml0, mr0, z0)
    t1, er1, ml1, mr1 = _tc23(p00, p01, bias0.reshape(1, IN_DIM),
                              gamma.reshape(1, IN_DIM),
                              beta.reshape(1, IN_DIM), w1p, alp1, arp1)
    z1 = jnp.zeros((N, 64), F32)
    p10, p11 = _edge_pass1(edge_index, t1, er1, ml1, mr1, z1)
    return _tc4(p10, p11, bias1.reshape(1, C))
